# baseline (speedup 1.0000x reference)
import functools

import jax
import jax.numpy as jnp
from jax import lax
from jax.experimental import pallas as pl
from jax.experimental.pallas import tpu as pltpu
from jax.experimental.pallas import tpu_sc as plsc

N_ROWS = 100000
N_J = 64
N_C = 32
N_R = N_J * N_C
B_TOT = 16384
NC, NS, L = 2, 16, 16
NW = NC * NS
BPW = B_TOT // NW
GRP = 16
NGRP = BPW // GRP
NBUF = 6
WIN = 128


def _sc_gather_kernel(t2, b32, c32):
    mesh = plsc.VectorSubcoreMesh(core_axis_name="core", subcore_axis_name="sub",
                                  num_cores=NC, num_subcores=NS)

    @functools.partial(
        pl.kernel,
        out_type=jax.ShapeDtypeStruct((B_TOT, N_J), jnp.float32),
        mesh=mesh,
        compiler_params=pltpu.CompilerParams(use_tc_tiling_on_sc=True,
                                             needs_layout_passes=False),
        scratch_types=[
            pltpu.VMEM((BPW,), jnp.int32),
            pltpu.VMEM((BPW,), jnp.int32),
            pltpu.VMEM((NBUF, N_J, WIN), jnp.float32),
            pltpu.VMEM((BPW, N_J), jnp.float32),
            pltpu.SemaphoreType.DMA,
            pltpu.SemaphoreType.DMA,
        ] + [pltpu.SemaphoreType.DMA] * NBUF,
    )
    def k(t_hbm, b_hbm, c_hbm, out_hbm, bidx, cidx, slabs, outb,
          sem_i, sem_o, *sems):
        wid = lax.axis_index("sub") * NC + lax.axis_index("core")
        base = wid * BPW
        cp_b = pltpu.async_copy(b_hbm.at[pl.ds(base, BPW)], bidx, sem_i)
        cp_c = pltpu.async_copy(c_hbm.at[pl.ds(base, BPW)], cidx, sem_i)
        cp_b.wait()
        cp_c.wait()

        lane = lax.iota(jnp.int32, L)

        def fire(ci_scalar, bi_scalar, q):
            b0 = pl.multiple_of((bi_scalar // WIN) * WIN, WIN)
            return pltpu.async_copy(
                t_hbm.at[:, ci_scalar, pl.ds(b0, WIN)], slabs.at[q], sems[q])

        def extract(i_glob, bm_scalar, q):
            col = jnp.full((L,), bm_scalar, jnp.int32)
            irow = jnp.full((L,), i_glob, jnp.int32)
            for h in range(N_J // L):
                row = h * L + lane
                vals = plsc.load_gather(slabs.at[q], [row, col])
                plsc.store_scatter(outb, [irow, row], vals)

        def group(g, _):
            bvec = bidx[pl.ds(g * GRP, GRP)]
            cvec = cidx[pl.ds(g * GRP, GRP)]
            pend = {}
            for k_ in range(GRP):
                q = k_ % NBUF
                if k_ >= NBUF:
                    i_prev = k_ - NBUF
                    pend.pop(i_prev).wait()
                    bm = bvec[i_prev] % WIN
                    extract(g * GRP + i_prev, bm, q)
                pend[k_] = fire(cvec[k_], bvec[k_], q)
            for k_ in range(GRP - NBUF, GRP):
                q = k_ % NBUF
                pend.pop(k_).wait()
                bm = bvec[k_] % WIN
                extract(g * GRP + k_, bm, q)
            return 0

        lax.fori_loop(0, NGRP, group, 0)

        pltpu.sync_copy(outb, out_hbm.at[pl.ds(base, BPW)])

    return k(t2, b32, c32)


def kernel(a, b, c):
    t = jnp.transpose(a, (1, 2, 0))
    b32 = b.astype(jnp.int32)
    c32 = c.astype(jnp.int32)
    return _sc_gather_kernel(t, b32, c32)


# speedup vs baseline: 1.0451x; 1.0451x over previous
import functools

import jax
import jax.numpy as jnp
from jax import lax
from jax.experimental import pallas as pl
from jax.experimental.pallas import tpu as pltpu
from jax.experimental.pallas import tpu_sc as plsc

N_ROWS = 100000
N_J = 64
N_C = 32
N_R = N_J * N_C
B_TOT = 16384
NC, NS, L = 2, 16, 16
NW = NC * NS
BPW = B_TOT // NW
GRP = 32
NGRP = BPW // GRP
NBUF = 6
WIN = 128


def _sc_gather_kernel(t2, b32, c32):
    mesh = plsc.VectorSubcoreMesh(core_axis_name="core", subcore_axis_name="sub",
                                  num_cores=NC, num_subcores=NS)

    @functools.partial(
        pl.kernel,
        out_type=jax.ShapeDtypeStruct((B_TOT, N_J), jnp.float32),
        mesh=mesh,
        compiler_params=pltpu.CompilerParams(use_tc_tiling_on_sc=True,
                                             needs_layout_passes=False),
        scratch_types=[
            pltpu.VMEM((BPW,), jnp.int32),
            pltpu.VMEM((BPW,), jnp.int32),
            pltpu.VMEM((NBUF, N_J, WIN), jnp.float32),
            pltpu.VMEM((BPW, N_J), jnp.float32),
            pltpu.SemaphoreType.DMA,
            pltpu.SemaphoreType.DMA,
        ] + [pltpu.SemaphoreType.DMA] * NBUF,
    )
    def k(t_hbm, b_hbm, c_hbm, out_hbm, bidx, cidx, slabs, outb,
          sem_i, sem_o, *sems):
        wid = lax.axis_index("sub") * NC + lax.axis_index("core")
        base = wid * BPW
        cp_b = pltpu.async_copy(b_hbm.at[pl.ds(base, BPW)], bidx, sem_i)
        cp_c = pltpu.async_copy(c_hbm.at[pl.ds(base, BPW)], cidx, sem_i)
        cp_b.wait()
        cp_c.wait()

        lane = lax.iota(jnp.int32, L)

        def fire(ci_scalar, bi_scalar, q):
            b0 = pl.multiple_of((bi_scalar // WIN) * WIN, WIN)
            return pltpu.async_copy(
                t_hbm.at[:, ci_scalar, pl.ds(b0, WIN)], slabs.at[q], sems[q])

        def extract(i_glob, bm_scalar, q):
            col = jnp.full((L,), bm_scalar, jnp.int32)
            irow = jnp.full((L,), i_glob, jnp.int32)
            for h in range(N_J // L):
                row = h * L + lane
                vals = plsc.load_gather(slabs.at[q], [row, col])
                plsc.store_scatter(outb, [irow, row], vals)

        def group(g, _):
            bv = [bidx[pl.ds(g * GRP + h * L, L)] for h in range(GRP // L)]
            cv = [cidx[pl.ds(g * GRP + h * L, L)] for h in range(GRP // L)]

            def bc(k_):
                return bv[k_ // L][k_ % L], cv[k_ // L][k_ % L]

            pend = {}
            for k_ in range(GRP):
                q = k_ % NBUF
                if k_ >= NBUF:
                    i_prev = k_ - NBUF
                    pend.pop(i_prev).wait()
                    bm = bc(i_prev)[0] % WIN
                    extract(g * GRP + i_prev, bm, q)
                bk, ck = bc(k_)
                pend[k_] = fire(ck, bk, q)
            for k_ in range(GRP - NBUF, GRP):
                q = k_ % NBUF
                pend.pop(k_).wait()
                bm = bc(k_)[0] % WIN
                extract(g * GRP + k_, bm, q)
            return 0

        lax.fori_loop(0, NGRP, group, 0)

        pltpu.sync_copy(outb, out_hbm.at[pl.ds(base, BPW)])

    return k(t2, b32, c32)


def kernel(a, b, c):
    t = jnp.transpose(a, (1, 2, 0))
    b32 = b.astype(jnp.int32)
    c32 = c.astype(jnp.int32)
    return _sc_gather_kernel(t, b32, c32)


# speedup vs baseline: 1.1125x; 1.0644x over previous
import functools

import jax
import jax.numpy as jnp
from jax import lax
from jax.experimental import pallas as pl
from jax.experimental.pallas import tpu as pltpu
from jax.experimental.pallas import tpu_sc as plsc

N_ROWS = 100000
N_J = 64
N_C = 32
N_R = N_J * N_C
B_TOT = 16384
NC, NS, L = 2, 16, 16
NW = NC * NS
BPW = B_TOT // NW
GRP = 32
NGRP = BPW // GRP
NBUF = 12
WIN = 128


def _sc_gather_kernel(t2, b32, c32):
    mesh = plsc.VectorSubcoreMesh(core_axis_name="core", subcore_axis_name="sub",
                                  num_cores=NC, num_subcores=NS)

    @functools.partial(
        pl.kernel,
        out_type=jax.ShapeDtypeStruct((B_TOT, N_J), jnp.float32),
        mesh=mesh,
        compiler_params=pltpu.CompilerParams(use_tc_tiling_on_sc=True,
                                             needs_layout_passes=False),
        scratch_types=[
            pltpu.VMEM((BPW,), jnp.int32),
            pltpu.VMEM((BPW,), jnp.int32),
            pltpu.VMEM((NBUF, N_J, WIN), jnp.float32),
            pltpu.VMEM((2, GRP, N_J), jnp.float32),
            pltpu.SemaphoreType.DMA,
            pltpu.SemaphoreType.DMA,
        ] + [pltpu.SemaphoreType.DMA] * NBUF,
    )
    def k(t_hbm, b_hbm, c_hbm, out_hbm, bidx, cidx, slabs, outb,
          sem_i, sem_o, *sems):
        wid = lax.axis_index("sub") * NC + lax.axis_index("core")
        base = wid * BPW
        cp_b = pltpu.async_copy(b_hbm.at[pl.ds(base, BPW)], bidx, sem_i)
        cp_c = pltpu.async_copy(c_hbm.at[pl.ds(base, BPW)], cidx, sem_i)
        cp_b.wait()
        cp_c.wait()

        lane = lax.iota(jnp.int32, L)

        def fire(ci_scalar, bi_scalar, q):
            b0 = pl.multiple_of((bi_scalar // WIN) * WIN, WIN)
            return pltpu.async_copy(
                t_hbm.at[:, ci_scalar, pl.ds(b0, WIN)], slabs.at[q], sems[q])

        def extract(k_loc, og, bm_scalar, q):
            col = jnp.full((L,), bm_scalar, jnp.int32)
            irow = jnp.full((L,), k_loc, jnp.int32)
            for h in range(N_J // L):
                row = h * L + lane
                vals = plsc.load_gather(slabs.at[q], [row, col])
                plsc.store_scatter(outb.at[og], [irow, row], vals)

        def group(g, _):
            og = g % 2
            bv = [bidx[pl.ds(g * GRP + h * L, L)] for h in range(GRP // L)]
            cv = [cidx[pl.ds(g * GRP + h * L, L)] for h in range(GRP // L)]

            def bc(k_):
                return bv[k_ // L][k_ % L], cv[k_ // L][k_ % L]

            @pl.when(g >= 2)
            def _():
                pltpu.make_async_copy(
                    outb.at[og], out_hbm.at[pl.ds(base, GRP)], sem_o).wait()

            pend = {}
            for k_ in range(GRP):
                q = k_ % NBUF
                if k_ >= NBUF:
                    i_prev = k_ - NBUF
                    pend.pop(i_prev).wait()
                    bm = bc(i_prev)[0] % WIN
                    extract(i_prev, og, bm, q)
                bk, ck = bc(k_)
                pend[k_] = fire(ck, bk, q)
            for k_ in range(GRP - NBUF, GRP):
                q = k_ % NBUF
                pend.pop(k_).wait()
                bm = bc(k_)[0] % WIN
                extract(k_, og, bm, q)

            pltpu.async_copy(
                outb.at[og], out_hbm.at[pl.ds(base + g * GRP, GRP)], sem_o)
            return 0

        lax.fori_loop(0, NGRP, group, 0)
        for _ in range(2):
            pltpu.make_async_copy(
                outb.at[0], out_hbm.at[pl.ds(base, GRP)], sem_o).wait()

    return k(t2, b32, c32)


def kernel(a, b, c):
    t = jnp.transpose(a, (1, 2, 0))
    b32 = b.astype(jnp.int32)
    c32 = c.astype(jnp.int32)
    return _sc_gather_kernel(t, b32, c32)


# speedup vs baseline: 1.1416x; 1.0261x over previous
import functools

import jax
import jax.numpy as jnp
from jax import lax
from jax.experimental import pallas as pl
from jax.experimental.pallas import tpu as pltpu
from jax.experimental.pallas import tpu_sc as plsc

N_ROWS = 100000
N_J = 64
N_C = 32
N_R = N_J * N_C
B_TOT = 16384
NC, NS, L = 2, 16, 16
NW = NC * NS
BPW = B_TOT // NW
GRP = 64
NGRP = BPW // GRP
NBUF = 13
WIN = 128


def _sc_gather_kernel(t2, b32, c32):
    mesh = plsc.VectorSubcoreMesh(core_axis_name="core", subcore_axis_name="sub",
                                  num_cores=NC, num_subcores=NS)

    @functools.partial(
        pl.kernel,
        out_type=jax.ShapeDtypeStruct((B_TOT, N_J), jnp.float32),
        mesh=mesh,
        compiler_params=pltpu.CompilerParams(use_tc_tiling_on_sc=True,
                                             needs_layout_passes=False),
        scratch_types=[
            pltpu.VMEM((BPW,), jnp.int32),
            pltpu.VMEM((BPW,), jnp.int32),
            pltpu.VMEM((NBUF, N_J, WIN), jnp.float32),
            pltpu.VMEM((2, GRP, N_J), jnp.float32),
            pltpu.SemaphoreType.DMA,
            pltpu.SemaphoreType.DMA,
        ] + [pltpu.SemaphoreType.DMA] * NBUF,
    )
    def k(t_hbm, b_hbm, c_hbm, out_hbm, bidx, cidx, slabs, outb,
          sem_i, sem_o, *sems):
        wid = lax.axis_index("sub") * NC + lax.axis_index("core")
        base = wid * BPW
        cp_b = pltpu.async_copy(b_hbm.at[pl.ds(base, BPW)], bidx, sem_i)
        cp_c = pltpu.async_copy(c_hbm.at[pl.ds(base, BPW)], cidx, sem_i)
        cp_b.wait()
        cp_c.wait()

        lane = lax.iota(jnp.int32, L)

        def fire(ci_scalar, bi_scalar, q):
            b0 = pl.multiple_of((bi_scalar // WIN) * WIN, WIN)
            return pltpu.async_copy(
                t_hbm.at[:, ci_scalar, pl.ds(b0, WIN)], slabs.at[q], sems[q])

        def extract(k_loc, og, bm_scalar, q):
            col = jnp.full((L,), bm_scalar, jnp.int32)
            irow = jnp.full((L,), k_loc, jnp.int32)
            for h in range(N_J // L):
                row = h * L + lane
                vals = plsc.load_gather(slabs.at[q], [row, col])
                plsc.store_scatter(outb.at[og], [irow, row], vals)

        def group(g, _):
            og = g % 2
            bv = [bidx[pl.ds(g * GRP + h * L, L)] for h in range(GRP // L)]
            cv = [cidx[pl.ds(g * GRP + h * L, L)] for h in range(GRP // L)]

            def bc(k_):
                return bv[k_ // L][k_ % L], cv[k_ // L][k_ % L]

            @pl.when(g >= 2)
            def _():
                pltpu.make_async_copy(
                    outb.at[og], out_hbm.at[pl.ds(base, GRP)], sem_o).wait()

            pend = {}
            for k_ in range(GRP):
                q = k_ % NBUF
                if k_ >= NBUF:
                    i_prev = k_ - NBUF
                    pend.pop(i_prev).wait()
                    bm = bc(i_prev)[0] % WIN
                    extract(i_prev, og, bm, q)
                bk, ck = bc(k_)
                pend[k_] = fire(ck, bk, q)
            for k_ in range(GRP - NBUF, GRP):
                q = k_ % NBUF
                pend.pop(k_).wait()
                bm = bc(k_)[0] % WIN
                extract(k_, og, bm, q)

            pltpu.async_copy(
                outb.at[og], out_hbm.at[pl.ds(base + g * GRP, GRP)], sem_o)
            return 0

        lax.fori_loop(0, NGRP, group, 0)
        for _ in range(2):
            pltpu.make_async_copy(
                outb.at[0], out_hbm.at[pl.ds(base, GRP)], sem_o).wait()

    return k(t2, b32, c32)


def kernel(a, b, c):
    t = jnp.transpose(a, (1, 2, 0))
    b32 = b.astype(jnp.int32)
    c32 = c.astype(jnp.int32)
    return _sc_gather_kernel(t, b32, c32)


# speedup vs baseline: 1.1572x; 1.0137x over previous
import functools

import jax
import jax.numpy as jnp
from jax import lax
from jax.experimental import pallas as pl
from jax.experimental.pallas import tpu as pltpu
from jax.experimental.pallas import tpu_sc as plsc

N_ROWS = 100000
N_J = 64
N_C = 32
N_R = N_J * N_C
B_TOT = 16384
NC, NS, L = 2, 16, 16
NW = NC * NS
BPW = B_TOT // NW
GRP = 128
NGRP = BPW // GRP
NBUF = 11
WIN = 128


def _sc_gather_kernel(t2, b32, c32):
    mesh = plsc.VectorSubcoreMesh(core_axis_name="core", subcore_axis_name="sub",
                                  num_cores=NC, num_subcores=NS)

    @functools.partial(
        pl.kernel,
        out_type=jax.ShapeDtypeStruct((B_TOT, N_J), jnp.float32),
        mesh=mesh,
        compiler_params=pltpu.CompilerParams(use_tc_tiling_on_sc=True,
                                             needs_layout_passes=False),
        scratch_types=[
            pltpu.VMEM((BPW,), jnp.int32),
            pltpu.VMEM((BPW,), jnp.int32),
            pltpu.VMEM((NBUF, N_J, WIN), jnp.float32),
            pltpu.VMEM((2, GRP, N_J), jnp.float32),
            pltpu.SemaphoreType.DMA,
            pltpu.SemaphoreType.DMA,
        ] + [pltpu.SemaphoreType.DMA] * NBUF,
    )
    def k(t_hbm, b_hbm, c_hbm, out_hbm, bidx, cidx, slabs, outb,
          sem_i, sem_o, *sems):
        wid = lax.axis_index("sub") * NC + lax.axis_index("core")
        base = wid * BPW
        cp_b = pltpu.async_copy(b_hbm.at[pl.ds(base, BPW)], bidx, sem_i)
        cp_c = pltpu.async_copy(c_hbm.at[pl.ds(base, BPW)], cidx, sem_i)
        cp_b.wait()
        cp_c.wait()

        lane = lax.iota(jnp.int32, L)

        def fire(ci_scalar, bi_scalar, q):
            b0 = pl.multiple_of((bi_scalar // WIN) * WIN, WIN)
            return pltpu.async_copy(
                t_hbm.at[:, ci_scalar, pl.ds(b0, WIN)], slabs.at[q], sems[q])

        def extract(k_loc, og, bm_scalar, q):
            col = jnp.full((L,), bm_scalar, jnp.int32)
            irow = jnp.full((L,), k_loc, jnp.int32)
            for h in range(N_J // L):
                row = h * L + lane
                vals = plsc.load_gather(slabs.at[q], [row, col])
                plsc.store_scatter(outb.at[og], [irow, row], vals)

        def group(g, _):
            og = g % 2
            bv = [bidx[pl.ds(g * GRP + h * L, L)] for h in range(GRP // L)]
            cv = [cidx[pl.ds(g * GRP + h * L, L)] for h in range(GRP // L)]

            def bc(k_):
                return bv[k_ // L][k_ % L], cv[k_ // L][k_ % L]

            @pl.when(g >= 2)
            def _():
                pltpu.make_async_copy(
                    outb.at[og], out_hbm.at[pl.ds(base, GRP)], sem_o).wait()

            pend = {}
            for k_ in range(GRP):
                q = k_ % NBUF
                if k_ >= NBUF:
                    i_prev = k_ - NBUF
                    pend.pop(i_prev).wait()
                    bm = bc(i_prev)[0] % WIN
                    extract(i_prev, og, bm, q)
                bk, ck = bc(k_)
                pend[k_] = fire(ck, bk, q)
            for k_ in range(GRP - NBUF, GRP):
                q = k_ % NBUF
                pend.pop(k_).wait()
                bm = bc(k_)[0] % WIN
                extract(k_, og, bm, q)

            pltpu.async_copy(
                outb.at[og], out_hbm.at[pl.ds(base + g * GRP, GRP)], sem_o)
            return 0

        lax.fori_loop(0, NGRP, group, 0)
        for _ in range(2):
            pltpu.make_async_copy(
                outb.at[0], out_hbm.at[pl.ds(base, GRP)], sem_o).wait()

    return k(t2, b32, c32)


def kernel(a, b, c):
    t = jnp.transpose(a, (1, 2, 0))
    b32 = b.astype(jnp.int32)
    c32 = c.astype(jnp.int32)
    return _sc_gather_kernel(t, b32, c32)
